# baseline (speedup 1.0000x reference)
import dataclasses
import functools

import jax
import jax.numpy as jnp
from jax import lax
from jax.experimental import pallas as pl
from jax.experimental.pallas import tpu as pltpu
from jax.experimental.pallas import tpu_sc as plsc

N = 10000
E = 320000
DH = 128
NCLS = 40
T = 1024
D1 = 144
D2 = 48
NC = 2
NS = 16
NW = NC * NS
EPW = E // NW
EB = 80
NBLK = EPW // EB
RPS = 632
NP = RPS * NS
TPS = T // NS

_HIGHEST = lax.Precision.HIGHEST


def _wrep_body(w_ref, o_ref):
    o_ref[...] = jnp.broadcast_to(w_ref[...], (w_ref.shape[0], 16))


def _mm1_body(f_ref, w_ref, o_ref):
    x = jnp.dot(f_ref[...], w_ref[...], preferred_element_type=jnp.float32,
                precision=_HIGHEST)
    col = lax.broadcasted_iota(jnp.int32, (x.shape[0], D1 - DH), 1)
    extra = jnp.where(col == 0, 1.0, 0.0).astype(jnp.float32)
    o_ref[...] = jnp.concatenate([x, extra], axis=1)


def _mid_body(p0_ref, p1_ref, b1_ref, w2_ref, o_ref):
    a = p0_ref[...] + p1_ref[...]
    agg = a[:, :DH]
    deg = a[:, DH:DH + 1]
    h = jnp.maximum(agg / jnp.maximum(deg, 1.0) + b1_ref[...], 0.0)
    x2 = jnp.dot(h, w2_ref[...], preferred_element_type=jnp.float32,
                 precision=_HIGHEST)
    col = lax.broadcasted_iota(jnp.int32, x2.shape, 1)
    o_ref[...] = jnp.where(col == NCLS, 1.0, x2)


def _loss_body(g0_ref, g1_ref, lab_ref, b2_ref, logits_ref, loss_ref):
    a = g0_ref[...] + g1_ref[...]
    deg = a[:, NCLS:NCLS + 1]
    logits = a[:, :NCLS] / jnp.maximum(deg, 1.0) + b2_ref[...]
    m = jnp.max(logits, axis=-1, keepdims=True)
    lse = m + jnp.log(jnp.sum(jnp.exp(logits - m), axis=-1, keepdims=True))
    lp = logits - lse
    losses = -jnp.sum(lab_ref[...] * lp, axis=-1, keepdims=True)
    logits_ref[...] = logits
    loss_ref[...] = jnp.mean(losses).reshape(1, 1)


def _make_agg(D, gather_targets):
    mesh = plsc.VectorSubcoreMesh(core_axis_name="c", subcore_axis_name="s",
                                  num_cores=NC, num_subcores=NS)
    parts_ty = jax.ShapeDtypeStruct((NC, N, D), jnp.float32)
    if gather_targets:
        out_type = (parts_ty, jax.ShapeDtypeStruct((NC, T, D), jnp.float32))
    else:
        out_type = parts_ty
    NSLOT = 2
    slot_scratch = []
    for _sl in range(NSLOT):
        slot_scratch += [
            pltpu.VMEM((EB,), jnp.int32),
            pltpu.VMEM((EB,), jnp.int32),
            pltpu.VMEM((EB, 16), jnp.float32),
            pltpu.VMEM((EB, D), jnp.float32),
            pltpu.SemaphoreType.DMA,
            pltpu.SemaphoreType.DMA,
            pltpu.SemaphoreType.DMA,
        ]
    scratch = slot_scratch + [
        pltpu.VMEM_SHARED((NP, D), jnp.float32),
    ]
    if gather_targets:
        scratch += [pltpu.VMEM((TPS,), jnp.int32),
                    pltpu.VMEM((TPS, D), jnp.float32)]

    def body(x_hbm, dst_hbm, src_hbm, w_hbm, *rest):
        if gather_targets:
            (tgt_hbm, parts_hbm, out_hbm, *scr) = rest
            tgt_v, trows_v = scr[-2:]
            acc_sh = scr[-3]
        else:
            (parts_hbm, *scr) = rest
            acc_sh = scr[-1]
        slots = tuple(tuple(scr[7 * i:7 * i + 7]) for i in range(NSLOT))
        c = lax.axis_index("c")
        s = lax.axis_index("s")
        wid = s * NC + c
        zero = jnp.zeros((16,), jnp.float32)
        rows_v = slots[0][3]

        @pl.loop(0, EB)
        def _(r):
            for j in range(D // 16):
                rows_v[r, pl.ds(j * 16, 16)] = zero

        zbase = s * RPS

        @pl.loop(0, RPS // EB)
        def _(k):
            pltpu.sync_copy(rows_v, acc_sh.at[pl.ds(zbase + k * EB, EB)])

        pltpu.sync_copy(rows_v.at[pl.ds(0, RPS % EB)],
                        acc_sh.at[pl.ds(zbase + (RPS // EB) * EB, RPS % EB)])
        plsc.subcore_barrier()

        ebase = wid * EPW

        def prefetch(k, sl):
            dv, sv, wv, rv, _, gsem, _ = slots[sl]
            off = ebase + k * EB
            pltpu.sync_copy(dst_hbm.at[pl.ds(off, EB)], dv)
            pltpu.sync_copy(src_hbm.at[pl.ds(off, EB)], sv)
            pltpu.sync_copy(w_hbm.at[pl.ds(off, EB)], wv)
            pltpu.async_copy(x_hbm.at[dv], rv, gsem)

        def wait_gather(sl):
            dv, _, _, rv, _, gsem, _ = slots[sl]
            pltpu.make_async_copy(x_hbm.at[dv], rv, gsem).wait()

        def compute_scatter(sl):
            dv, sv, wv, rv, _, _, _ = slots[sl]

            @pl.loop(0, EB)
            def _(e):
                wvec = wv[e, pl.ds(0, 16)]
                for j in range(D // 16):
                    slc = pl.ds(j * 16, 16)
                    rv[e, slc] = rv[e, slc] * wvec

            pltpu.sync_copy(rv, acc_sh.at[sv], add=True)

        prefetch(0, 0)

        @pl.loop(0, NBLK // 2)
        def _(i):
            wait_gather(0)
            prefetch(2 * i + 1, 1)
            compute_scatter(0)
            wait_gather(1)

            @pl.when(2 * i + 2 < NBLK)
            def _():
                prefetch(2 * i + 2, 0)

            compute_scatter(1)

        wait_gather(0)
        compute_scatter(0)
        plsc.subcore_barrier()

        @pl.when(s != NS - 1)
        def _():
            @pl.loop(0, 7)
            def _(k):
                r0 = zbase + k * EB
                pltpu.sync_copy(acc_sh.at[pl.ds(r0, EB)],
                                parts_hbm.at[c].at[pl.ds(r0, EB)])
            r0 = zbase + 7 * EB
            pltpu.sync_copy(acc_sh.at[pl.ds(r0, 72)],
                            parts_hbm.at[c].at[pl.ds(r0, 72)])

        @pl.when(s == NS - 1)
        def _():
            @pl.loop(0, 6)
            def _(k):
                r0 = zbase + k * EB
                pltpu.sync_copy(acc_sh.at[pl.ds(r0, EB)],
                                parts_hbm.at[c].at[pl.ds(r0, EB)])
            r0 = zbase + 6 * EB
            pltpu.sync_copy(acc_sh.at[pl.ds(r0, 40)],
                            parts_hbm.at[c].at[pl.ds(r0, 40)])

        if gather_targets:
            plsc.subcore_barrier()
            tbase = s * TPS
            pltpu.sync_copy(tgt_hbm.at[pl.ds(tbase, TPS)], tgt_v)
            pltpu.async_copy(parts_hbm.at[c].at[tgt_v], trows_v,
                             slots[0][5]).wait()
            pltpu.sync_copy(trows_v, out_hbm.at[c].at[pl.ds(tbase, TPS)])

    cp = pltpu.CompilerParams(needs_layout_passes=False,
                              use_tc_tiling_on_sc=False)
    return pl.kernel(body, out_type=out_type, mesh=mesh,
                     scratch_types=scratch, compiler_params=cp)


_agg1 = _make_agg(D1, gather_targets=False)
_agg2 = _make_agg(D2, gather_targets=True)


def kernel(features, edge_srcs, edge_dsts, edge_weights, targets, labels,
           W1, b1, W2, b2):
    f32 = jnp.float32
    GB = 1000

    WB = 2000
    wrep = pl.pallas_call(
        _wrep_body,
        grid=(E // WB,),
        in_specs=[pl.BlockSpec((WB, 1), lambda i: (i, 0))],
        out_specs=pl.BlockSpec((WB, 16), lambda i: (i, 0)),
        out_shape=jax.ShapeDtypeStruct((E, 16), f32),
    )(edge_weights.reshape(E, 1))

    x1 = pl.pallas_call(
        _mm1_body,
        grid=(N // GB,),
        in_specs=[
            pl.BlockSpec((GB, DH), lambda i: (i, 0)),
            pl.BlockSpec((DH, DH), lambda i: (0, 0)),
        ],
        out_specs=pl.BlockSpec((GB, D1), lambda i: (i, 0)),
        out_shape=jax.ShapeDtypeStruct((N, D1), f32),
    )(features, W1)

    parts1 = _agg1(x1, edge_dsts, edge_srcs, wrep)

    w2pad = jnp.concatenate(
        [W2, jnp.zeros((DH, D2 - NCLS), f32)], axis=1)
    x2 = pl.pallas_call(
        _mid_body,
        grid=(N // GB,),
        in_specs=[
            pl.BlockSpec((GB, D1), lambda i: (i, 0)),
            pl.BlockSpec((GB, D1), lambda i: (i, 0)),
            pl.BlockSpec((1, DH), lambda i: (0, 0)),
            pl.BlockSpec((DH, D2), lambda i: (0, 0)),
        ],
        out_specs=pl.BlockSpec((GB, D2), lambda i: (i, 0)),
        out_shape=jax.ShapeDtypeStruct((N, D2), f32),
    )(parts1[0], parts1[1], b1.reshape(1, DH), w2pad)

    _, gath = _agg2(x2, edge_dsts, edge_srcs, wrep, targets)

    logits, loss = pl.pallas_call(
        _loss_body,
        in_specs=[
            pl.BlockSpec((T, D2), lambda: (0, 0)),
            pl.BlockSpec((T, D2), lambda: (0, 0)),
            pl.BlockSpec((T, NCLS), lambda: (0, 0)),
            pl.BlockSpec((1, NCLS), lambda: (0, 0)),
        ],
        out_specs=[
            pl.BlockSpec((T, NCLS), lambda: (0, 0)),
            pl.BlockSpec((1, 1), lambda: (0, 0)),
        ],
        out_shape=[
            jax.ShapeDtypeStruct((T, NCLS), f32),
            jax.ShapeDtypeStruct((1, 1), f32),
        ],
    )(gath[0], gath[1], labels, b2.reshape(1, NCLS))

    return (loss[0, 0], logits)


# speedup vs baseline: 2.8457x; 2.8457x over previous
import dataclasses
import functools

import jax
import jax.numpy as jnp
from jax import lax
from jax.experimental import pallas as pl
from jax.experimental.pallas import tpu as pltpu
from jax.experimental.pallas import tpu_sc as plsc

N = 10000
E = 320000
DH = 128
NCLS = 40
T = 1024
D1 = 144
D2 = 48
NC = 2
NS = 16
NW = NC * NS
EPW = E // NW
EB = 80
NBLK = EPW // EB
RPS = 632
NP = RPS * NS
TPS = T // NS

_HIGHEST = lax.Precision.HIGHEST


def _wrep_body(w_ref, o_ref):
    o_ref[...] = jnp.broadcast_to(w_ref[...], (w_ref.shape[0], 16))


def _mm1_body(f_ref, w_ref, o_ref):
    x = jnp.dot(f_ref[...], w_ref[...], preferred_element_type=jnp.float32,
                precision=_HIGHEST)
    col = lax.broadcasted_iota(jnp.int32, (x.shape[0], D1 - DH), 1)
    extra = jnp.where(col == 0, 1.0, 0.0).astype(jnp.float32)
    o_ref[...] = jnp.concatenate([x, extra], axis=1)


def _mid_body(p0_ref, p1_ref, b1_ref, w2_ref, o_ref):
    a = p0_ref[...] + p1_ref[...]
    agg = a[:, :DH]
    deg = a[:, DH:DH + 1]
    h = jnp.maximum(agg / jnp.maximum(deg, 1.0) + b1_ref[...], 0.0)
    x2 = jnp.dot(h, w2_ref[...], preferred_element_type=jnp.float32,
                 precision=_HIGHEST)
    col = lax.broadcasted_iota(jnp.int32, x2.shape, 1)
    o_ref[...] = jnp.where(col == NCLS, 1.0, x2)


def _loss_body(g0_ref, g1_ref, lab_ref, b2_ref, logits_ref, loss_ref):
    a = g0_ref[...] + g1_ref[...]
    deg = a[:, NCLS:NCLS + 1]
    logits = a[:, :NCLS] / jnp.maximum(deg, 1.0) + b2_ref[...]
    m = jnp.max(logits, axis=-1, keepdims=True)
    lse = m + jnp.log(jnp.sum(jnp.exp(logits - m), axis=-1, keepdims=True))
    lp = logits - lse
    losses = -jnp.sum(lab_ref[...] * lp, axis=-1, keepdims=True)
    logits_ref[...] = logits
    loss_ref[...] = jnp.mean(losses).reshape(1, 1)


def _make_agg(D, gather_targets):
    mesh = plsc.VectorSubcoreMesh(core_axis_name="c", subcore_axis_name="s",
                                  num_cores=NC, num_subcores=NS)
    parts_ty = jax.ShapeDtypeStruct((NC, N, D), jnp.float32)
    if gather_targets:
        out_type = (parts_ty, jax.ShapeDtypeStruct((NC, T, D), jnp.float32))
    else:
        out_type = parts_ty
    NSLOT = 2
    scratch = [
        pltpu.VMEM((EPW,), jnp.float32),
        pltpu.SemaphoreType.DMA,
    ]
    for _sl in range(NSLOT):
        scratch += [
            pltpu.VMEM((EB,), jnp.int32),
            pltpu.VMEM((EB,), jnp.int32),
            pltpu.VMEM((EB, D), jnp.float32),
            pltpu.SemaphoreType.DMA,
            pltpu.SemaphoreType.DMA,
            pltpu.SemaphoreType.DMA,
        ]
    scratch += [
        pltpu.VMEM_SHARED((NP, D), jnp.float32),
    ]
    if gather_targets:
        scratch += [pltpu.VMEM((TPS,), jnp.int32),
                    pltpu.VMEM((TPS, D), jnp.float32)]

    def body(x_hbm, dst_hbm, src_hbm, w_hbm, *rest):
        if gather_targets:
            (tgt_hbm, parts_hbm, out_hbm, *scr) = rest
            tgt_v, trows_v = scr[-2:]
            acc_sh = scr[-3]
        else:
            (parts_hbm, *scr) = rest
            acc_sh = scr[-1]
        wv_all, stg_sem = scr[0:2]
        slots = tuple(tuple(scr[2 + 6 * i:8 + 6 * i]) for i in range(NSLOT))
        c = lax.axis_index("c")
        s = lax.axis_index("s")
        wid = s * NC + c
        zero = jnp.zeros((16,), jnp.float32)
        zi = jnp.zeros((16,), jnp.int32)
        rows_v = slots[0][2]
        ebase = wid * EPW

        pltpu.async_copy(w_hbm.at[pl.ds(ebase, EPW)], wv_all, stg_sem)

        @pl.loop(0, EB)
        def _(r):
            for j in range(D // 16):
                rows_v[r, pl.ds(j * 16, 16)] = zero

        zbase = s * RPS

        @pl.loop(0, RPS // EB)
        def _(k):
            pltpu.sync_copy(rows_v, acc_sh.at[pl.ds(zbase + k * EB, EB)])

        pltpu.sync_copy(rows_v.at[pl.ds(0, RPS % EB)],
                        acc_sh.at[pl.ds(zbase + (RPS // EB) * EB, RPS % EB)])
        pltpu.make_async_copy(w_hbm.at[pl.ds(ebase, EPW)], wv_all,
                              stg_sem).wait()
        plsc.subcore_barrier()

        def issue_dst(k, sl):
            dv = slots[sl][0]
            dsem = slots[sl][3]
            pltpu.async_copy(dst_hbm.at[pl.ds(ebase + k * EB, EB)], dv, dsem)

        def prefetch(k, sl):
            dv, sv, rv, dsem, isem, gsem = slots[sl]
            off = ebase + k * EB
            pltpu.make_async_copy(dst_hbm.at[pl.ds(off, EB)], dv, dsem).wait()
            pltpu.async_copy(src_hbm.at[pl.ds(off, EB)], sv, isem)
            pltpu.async_copy(x_hbm.at[dv], rv, gsem)

        def wait_gather(sl):
            dv, _, rv, _, _, gsem = slots[sl]
            pltpu.make_async_copy(x_hbm.at[dv], rv, gsem).wait()

        def compute_scatter(k, sl):
            _, sv, rv, _, isem, _ = slots[sl]
            ok0 = k * EB

            @pl.loop(0, EB)
            def _(e):
                wvec = plsc.load_gather(wv_all, [zi + (ok0 + e)])
                for j in range(D // 16):
                    slc = pl.ds(j * 16, 16)
                    rv[e, slc] = rv[e, slc] * wvec

            off = ebase + ok0
            pltpu.make_async_copy(src_hbm.at[pl.ds(off, EB)], sv, isem).wait()
            pltpu.sync_copy(rv, acc_sh.at[sv], add=True)

        issue_dst(0, 0)
        prefetch(0, 0)
        issue_dst(1, 1)

        @pl.loop(0, NBLK // 2)
        def _(i):
            wait_gather(0)
            issue_dst(2 * i + 2, 0)
            prefetch(2 * i + 1, 1)
            compute_scatter(2 * i, 0)
            wait_gather(1)

            @pl.when(2 * i + 3 < NBLK)
            def _():
                issue_dst(2 * i + 3, 1)

            prefetch(2 * i + 2, 0)
            compute_scatter(2 * i + 1, 1)

        wait_gather(0)
        compute_scatter(NBLK - 1, 0)
        plsc.subcore_barrier()

        @pl.when(s != NS - 1)
        def _():
            @pl.loop(0, 7)
            def _(k):
                r0 = zbase + k * EB
                pltpu.sync_copy(acc_sh.at[pl.ds(r0, EB)],
                                parts_hbm.at[c].at[pl.ds(r0, EB)])
            r0 = zbase + 7 * EB
            pltpu.sync_copy(acc_sh.at[pl.ds(r0, 72)],
                            parts_hbm.at[c].at[pl.ds(r0, 72)])

        @pl.when(s == NS - 1)
        def _():
            @pl.loop(0, 6)
            def _(k):
                r0 = zbase + k * EB
                pltpu.sync_copy(acc_sh.at[pl.ds(r0, EB)],
                                parts_hbm.at[c].at[pl.ds(r0, EB)])
            r0 = zbase + 6 * EB
            pltpu.sync_copy(acc_sh.at[pl.ds(r0, 40)],
                            parts_hbm.at[c].at[pl.ds(r0, 40)])

        if gather_targets:
            plsc.subcore_barrier()
            tbase = s * TPS
            pltpu.sync_copy(tgt_hbm.at[pl.ds(tbase, TPS)], tgt_v)
            pltpu.async_copy(parts_hbm.at[c].at[tgt_v], trows_v,
                             slots[0][3]).wait()
            pltpu.sync_copy(trows_v, out_hbm.at[c].at[pl.ds(tbase, TPS)])

    cp = pltpu.CompilerParams(needs_layout_passes=False,
                              use_tc_tiling_on_sc=False)
    return pl.kernel(body, out_type=out_type, mesh=mesh,
                     scratch_types=scratch, compiler_params=cp)


_agg1 = _make_agg(D1, gather_targets=False)
_agg2 = _make_agg(D2, gather_targets=True)


def kernel(features, edge_srcs, edge_dsts, edge_weights, targets, labels,
           W1, b1, W2, b2):
    f32 = jnp.float32
    GB = 1000

    x1 = pl.pallas_call(
        _mm1_body,
        grid=(N // GB,),
        in_specs=[
            pl.BlockSpec((GB, DH), lambda i: (i, 0)),
            pl.BlockSpec((DH, DH), lambda i: (0, 0)),
        ],
        out_specs=pl.BlockSpec((GB, D1), lambda i: (i, 0)),
        out_shape=jax.ShapeDtypeStruct((N, D1), f32),
    )(features, W1)

    parts1 = _agg1(x1, edge_dsts, edge_srcs, edge_weights)

    w2pad = jnp.concatenate(
        [W2, jnp.zeros((DH, D2 - NCLS), f32)], axis=1)
    x2 = pl.pallas_call(
        _mid_body,
        grid=(N // GB,),
        in_specs=[
            pl.BlockSpec((GB, D1), lambda i: (i, 0)),
            pl.BlockSpec((GB, D1), lambda i: (i, 0)),
            pl.BlockSpec((1, DH), lambda i: (0, 0)),
            pl.BlockSpec((DH, D2), lambda i: (0, 0)),
        ],
        out_specs=pl.BlockSpec((GB, D2), lambda i: (i, 0)),
        out_shape=jax.ShapeDtypeStruct((N, D2), f32),
    )(parts1[0], parts1[1], b1.reshape(1, DH), w2pad)

    _, gath = _agg2(x2, edge_dsts, edge_srcs, edge_weights, targets)

    logits, loss = pl.pallas_call(
        _loss_body,
        in_specs=[
            pl.BlockSpec((T, D2), lambda: (0, 0)),
            pl.BlockSpec((T, D2), lambda: (0, 0)),
            pl.BlockSpec((T, NCLS), lambda: (0, 0)),
            pl.BlockSpec((1, NCLS), lambda: (0, 0)),
        ],
        out_specs=[
            pl.BlockSpec((T, NCLS), lambda: (0, 0)),
            pl.BlockSpec((1, 1), lambda: (0, 0)),
        ],
        out_shape=[
            jax.ShapeDtypeStruct((T, NCLS), f32),
            jax.ShapeDtypeStruct((1, 1), f32),
        ],
    )(gath[0], gath[1], labels, b2.reshape(1, NCLS))

    return (loss[0, 0], logits)


# speedup vs baseline: 2.8506x; 1.0017x over previous
import dataclasses
import functools

import jax
import jax.numpy as jnp
from jax import lax
from jax.experimental import pallas as pl
from jax.experimental.pallas import tpu as pltpu
from jax.experimental.pallas import tpu_sc as plsc

N = 10000
E = 320000
DH = 128
NCLS = 40
T = 1024
D1 = 144
D2 = 48
NC = 2
NS = 16
NW = NC * NS
EPW = E // NW
EB = 80
NBLK = EPW // EB
RPS = 632
NP = RPS * NS
TPS = T // NS

_HIGHEST = lax.Precision.HIGHEST


def _wrep_body(w_ref, o_ref):
    o_ref[...] = jnp.broadcast_to(w_ref[...], (w_ref.shape[0], 16))


def _mm1_body(f_ref, w_ref, o_ref):
    x = jnp.dot(f_ref[...], w_ref[...], preferred_element_type=jnp.float32,
                precision=_HIGHEST)
    col = lax.broadcasted_iota(jnp.int32, (x.shape[0], D1 - DH), 1)
    extra = jnp.where(col == 0, 1.0, 0.0).astype(jnp.float32)
    o_ref[...] = jnp.concatenate([x, extra], axis=1)


def _mid_body(p0_ref, p1_ref, b1_ref, w2_ref, o_ref):
    a = p0_ref[...] + p1_ref[...]
    agg = a[:, :DH]
    deg = a[:, DH:DH + 1]
    h = jnp.maximum(agg / jnp.maximum(deg, 1.0) + b1_ref[...], 0.0)
    x2 = jnp.dot(h, w2_ref[...], preferred_element_type=jnp.float32,
                 precision=_HIGHEST)
    col = lax.broadcasted_iota(jnp.int32, x2.shape, 1)
    o_ref[...] = jnp.where(col == NCLS, 1.0, x2)


def _loss_body(g0_ref, g1_ref, lab_ref, b2_ref, logits_ref, loss_ref):
    a = g0_ref[...] + g1_ref[...]
    deg = a[:, NCLS:NCLS + 1]
    logits = a[:, :NCLS] / jnp.maximum(deg, 1.0) + b2_ref[...]
    m = jnp.max(logits, axis=-1, keepdims=True)
    lse = m + jnp.log(jnp.sum(jnp.exp(logits - m), axis=-1, keepdims=True))
    lp = logits - lse
    losses = -jnp.sum(lab_ref[...] * lp, axis=-1, keepdims=True)
    logits_ref[...] = logits
    loss_ref[...] = jnp.mean(losses).reshape(1, 1)


def _make_agg(D, gather_targets):
    mesh = plsc.VectorSubcoreMesh(core_axis_name="c", subcore_axis_name="s",
                                  num_cores=NC, num_subcores=NS)
    parts_ty = jax.ShapeDtypeStruct((NC, N, D), jnp.float32)
    if gather_targets:
        out_type = (parts_ty, jax.ShapeDtypeStruct((NC, T, D), jnp.float32))
    else:
        out_type = parts_ty
    NSLOT = 2
    scratch = [
        pltpu.VMEM((EPW,), jnp.float32),
        pltpu.SemaphoreType.DMA,
    ]
    for _sl in range(NSLOT):
        scratch += [
            pltpu.VMEM((EB,), jnp.int32),
            pltpu.VMEM((EB,), jnp.int32),
            pltpu.VMEM((EB, D), jnp.float32),
            pltpu.SemaphoreType.DMA,
            pltpu.SemaphoreType.DMA,
            pltpu.SemaphoreType.DMA,
            pltpu.SemaphoreType.DMA,
        ]
    scratch += [
        pltpu.VMEM_SHARED((NP, D), jnp.float32),
    ]
    if gather_targets:
        scratch += [pltpu.VMEM((TPS,), jnp.int32),
                    pltpu.VMEM((TPS, D), jnp.float32)]

    def body(x_hbm, dst_hbm, src_hbm, w_hbm, *rest):
        if gather_targets:
            (tgt_hbm, parts_hbm, out_hbm, *scr) = rest
            tgt_v, trows_v = scr[-2:]
            acc_sh = scr[-3]
        else:
            (parts_hbm, *scr) = rest
            acc_sh = scr[-1]
        wv_all, stg_sem = scr[0:2]
        slots = tuple(tuple(scr[2 + 7 * i:9 + 7 * i]) for i in range(NSLOT))
        c = lax.axis_index("c")
        s = lax.axis_index("s")
        wid = s * NC + c
        zero = jnp.zeros((16,), jnp.float32)
        zi = jnp.zeros((16,), jnp.int32)
        rows_v = slots[0][2]
        ebase = wid * EPW

        pltpu.async_copy(w_hbm.at[pl.ds(ebase, EPW)], wv_all, stg_sem)

        @pl.loop(0, EB)
        def _(r):
            for j in range(D // 16):
                rows_v[r, pl.ds(j * 16, 16)] = zero

        zbase = s * RPS

        @pl.loop(0, RPS // EB)
        def _(k):
            pltpu.sync_copy(rows_v, acc_sh.at[pl.ds(zbase + k * EB, EB)])

        pltpu.sync_copy(rows_v.at[pl.ds(0, RPS % EB)],
                        acc_sh.at[pl.ds(zbase + (RPS // EB) * EB, RPS % EB)])
        pltpu.make_async_copy(w_hbm.at[pl.ds(ebase, EPW)], wv_all,
                              stg_sem).wait()
        plsc.subcore_barrier()

        def issue_dst(k, sl):
            dv = slots[sl][0]
            dsem = slots[sl][3]
            pltpu.async_copy(dst_hbm.at[pl.ds(ebase + k * EB, EB)], dv, dsem)

        def wait_scatter(sl):
            rv, ssem = slots[sl][2], slots[sl][6]
            pltpu.make_async_copy(x_hbm.at[pl.ds(0, EB)], rv, ssem).wait()

        def prefetch(k, sl, drain=True):
            dv, sv, rv, dsem, isem, gsem, ssem = slots[sl]
            if drain:
                wait_scatter(sl)
            off = ebase + k * EB
            pltpu.make_async_copy(dst_hbm.at[pl.ds(off, EB)], dv, dsem).wait()
            pltpu.async_copy(src_hbm.at[pl.ds(off, EB)], sv, isem)
            pltpu.async_copy(x_hbm.at[dv], rv, gsem)

        def wait_gather(sl):
            dv, _, rv, _, _, gsem, _ = slots[sl]
            pltpu.make_async_copy(x_hbm.at[dv], rv, gsem).wait()

        def compute_scatter(k, sl):
            _, sv, rv, _, isem, _, ssem = slots[sl]
            ok0 = k * EB

            @pl.loop(0, EB)
            def _(e):
                wvec = plsc.load_gather(wv_all, [zi + (ok0 + e)])
                for j in range(D // 16):
                    slc = pl.ds(j * 16, 16)
                    rv[e, slc] = rv[e, slc] * wvec

            off = ebase + ok0
            pltpu.make_async_copy(src_hbm.at[pl.ds(off, EB)], sv, isem).wait()
            pltpu.async_copy(rv, acc_sh.at[sv], ssem, add=True)

        issue_dst(0, 0)
        prefetch(0, 0, drain=False)
        issue_dst(1, 1)

        @pl.loop(0, NBLK // 2)
        def _(i):
            wait_gather(0)
            issue_dst(2 * i + 2, 0)

            @pl.when(i == 0)
            def _():
                prefetch(1, 1, drain=False)

            @pl.when(i != 0)
            def _():
                prefetch(2 * i + 1, 1)

            compute_scatter(2 * i, 0)
            wait_gather(1)

            @pl.when(2 * i + 3 < NBLK)
            def _():
                issue_dst(2 * i + 3, 1)

            prefetch(2 * i + 2, 0)
            compute_scatter(2 * i + 1, 1)

        wait_gather(0)
        compute_scatter(NBLK - 1, 0)
        wait_scatter(0)
        wait_scatter(1)
        plsc.subcore_barrier()

        @pl.when(s != NS - 1)
        def _():
            @pl.loop(0, 7)
            def _(k):
                r0 = zbase + k * EB
                pltpu.sync_copy(acc_sh.at[pl.ds(r0, EB)],
                                parts_hbm.at[c].at[pl.ds(r0, EB)])
            r0 = zbase + 7 * EB
            pltpu.sync_copy(acc_sh.at[pl.ds(r0, 72)],
                            parts_hbm.at[c].at[pl.ds(r0, 72)])

        @pl.when(s == NS - 1)
        def _():
            @pl.loop(0, 6)
            def _(k):
                r0 = zbase + k * EB
                pltpu.sync_copy(acc_sh.at[pl.ds(r0, EB)],
                                parts_hbm.at[c].at[pl.ds(r0, EB)])
            r0 = zbase + 6 * EB
            pltpu.sync_copy(acc_sh.at[pl.ds(r0, 40)],
                            parts_hbm.at[c].at[pl.ds(r0, 40)])

        if gather_targets:
            plsc.subcore_barrier()
            tbase = s * TPS
            pltpu.sync_copy(tgt_hbm.at[pl.ds(tbase, TPS)], tgt_v)
            pltpu.async_copy(parts_hbm.at[c].at[tgt_v], trows_v,
                             slots[0][3]).wait()
            pltpu.sync_copy(trows_v, out_hbm.at[c].at[pl.ds(tbase, TPS)])

    cp = pltpu.CompilerParams(needs_layout_passes=False,
                              use_tc_tiling_on_sc=False)
    return pl.kernel(body, out_type=out_type, mesh=mesh,
                     scratch_types=scratch, compiler_params=cp)


_agg1 = _make_agg(D1, gather_targets=False)
_agg2 = _make_agg(D2, gather_targets=True)


def kernel(features, edge_srcs, edge_dsts, edge_weights, targets, labels,
           W1, b1, W2, b2):
    f32 = jnp.float32
    GB = 1000

    x1 = pl.pallas_call(
        _mm1_body,
        grid=(N // GB,),
        in_specs=[
            pl.BlockSpec((GB, DH), lambda i: (i, 0)),
            pl.BlockSpec((DH, DH), lambda i: (0, 0)),
        ],
        out_specs=pl.BlockSpec((GB, D1), lambda i: (i, 0)),
        out_shape=jax.ShapeDtypeStruct((N, D1), f32),
    )(features, W1)

    parts1 = _agg1(x1, edge_dsts, edge_srcs, edge_weights)

    w2pad = jnp.concatenate(
        [W2, jnp.zeros((DH, D2 - NCLS), f32)], axis=1)
    x2 = pl.pallas_call(
        _mid_body,
        grid=(N // GB,),
        in_specs=[
            pl.BlockSpec((GB, D1), lambda i: (i, 0)),
            pl.BlockSpec((GB, D1), lambda i: (i, 0)),
            pl.BlockSpec((1, DH), lambda i: (0, 0)),
            pl.BlockSpec((DH, D2), lambda i: (0, 0)),
        ],
        out_specs=pl.BlockSpec((GB, D2), lambda i: (i, 0)),
        out_shape=jax.ShapeDtypeStruct((N, D2), f32),
    )(parts1[0], parts1[1], b1.reshape(1, DH), w2pad)

    _, gath = _agg2(x2, edge_dsts, edge_srcs, edge_weights, targets)

    logits, loss = pl.pallas_call(
        _loss_body,
        in_specs=[
            pl.BlockSpec((T, D2), lambda: (0, 0)),
            pl.BlockSpec((T, D2), lambda: (0, 0)),
            pl.BlockSpec((T, NCLS), lambda: (0, 0)),
            pl.BlockSpec((1, NCLS), lambda: (0, 0)),
        ],
        out_specs=[
            pl.BlockSpec((T, NCLS), lambda: (0, 0)),
            pl.BlockSpec((1, 1), lambda: (0, 0)),
        ],
        out_shape=[
            jax.ShapeDtypeStruct((T, NCLS), f32),
            jax.ShapeDtypeStruct((1, 1), f32),
        ],
    )(gath[0], gath[1], labels, b2.reshape(1, NCLS))

    return (loss[0, 0], logits)
